# baseline (speedup 1.0000x reference)
import jax
import jax.numpy as jnp
from jax.experimental import pallas as pl

_NUM_FIELDS = 26
_CARD = 1000
_ROWS_PER_BLOCK = 512


def _onehot_body(xe_ref, out_ref):
    idx = xe_ref[...]
    iota = jax.lax.broadcasted_iota(jnp.int32, (_ROWS_PER_BLOCK, _CARD), 1)
    out_ref[...] = (iota == idx).astype(jnp.float32)


def kernel(xe):
    b, f = xe.shape
    rows = b * f
    xe_flat = xe.reshape(rows, 1)
    out2d = pl.pallas_call(
        _onehot_body,
        grid=(rows // _ROWS_PER_BLOCK,),
        in_specs=[pl.BlockSpec((_ROWS_PER_BLOCK, 1), lambda i: (i, 0))],
        out_specs=pl.BlockSpec((_ROWS_PER_BLOCK, _CARD), lambda i: (i, 0)),
        out_shape=jax.ShapeDtypeStruct((rows, _CARD), jnp.float32),
    )(xe_flat)
    return out2d.reshape(b, f * _CARD)


# speedup vs baseline: 1.5462x; 1.5462x over previous
import numpy as np
import jax
import jax.numpy as jnp
from jax.experimental import pallas as pl

_NUM_FIELDS = 26
_CARD = 1000
_OUT_COLS = _NUM_FIELDS * _CARD
_COL_BLOCK = 2048
_NUM_COL_BLOCKS = -(-_OUT_COLS // _COL_BLOCK)
_ROW_BLOCK = 512

_FIELD_IDX = np.minimum(
    (np.arange(_NUM_COL_BLOCKS) * _COL_BLOCK // _CARD)[:, None] + np.arange(3)[None, :],
    _NUM_FIELDS - 1,
)


def _onehot_body(tgt_ref, out_ref):
    j = pl.program_id(1)
    cols = j * _COL_BLOCK + jax.lax.broadcasted_iota(
        jnp.int32, (_ROW_BLOCK, _COL_BLOCK), 1
    )
    m = (
        (cols == tgt_ref[0, :, 0:1])
        | (cols == tgt_ref[0, :, 1:2])
        | (cols == tgt_ref[0, :, 2:3])
    )
    out_ref[...] = m.astype(jnp.float32)


def kernel(xe):
    b = xe.shape[0]
    fidx = jnp.asarray(_FIELD_IDX.reshape(-1), dtype=jnp.int32)
    tgt = jnp.take(xe, fidx, axis=1).astype(jnp.int32) + fidx * _CARD
    tgt = tgt.reshape(b, _NUM_COL_BLOCKS, 3).transpose(1, 0, 2)
    return pl.pallas_call(
        _onehot_body,
        grid=(b // _ROW_BLOCK, _NUM_COL_BLOCKS),
        in_specs=[pl.BlockSpec((1, _ROW_BLOCK, 3), lambda r, j: (j, r, 0))],
        out_specs=pl.BlockSpec((_ROW_BLOCK, _COL_BLOCK), lambda r, j: (r, j)),
        out_shape=jax.ShapeDtypeStruct((b, _OUT_COLS), jnp.float32),
    )(tgt)
